# Initial kernel scaffold; baseline (speedup 1.0000x reference)
import jax
import jax.numpy as jnp
from jax.experimental import pallas as pl


def kernel(a, b, attn_w, attn_b, fc1_w, fc2_w):
    raise NotImplementedError("write your pallas kernel here")


# speedup vs baseline: 15.4891x; 15.4891x over previous
import jax
import jax.numpy as jnp
from jax.experimental import pallas as pl
from jax.experimental.pallas import tpu as pltpu

_B, _C, _H, _W = 4, 96, 384, 384


def _pool_mlp_body(b_ref, fc1_ref, fc2_ref, o_ref):
    bi = pl.program_id(0)
    ci = pl.program_id(1)

    @pl.when((bi == 0) & (ci == 0))
    def _init():
        o_ref[...] = jnp.zeros_like(o_ref)

    x = b_ref[0, 0, :, :]
    rpar = jax.lax.broadcasted_iota(jnp.int32, (_H, _W), 0) % 2
    cpar = jax.lax.broadcasted_iota(jnp.int32, (_H, _W), 1) % 2
    s = jnp.sum(jnp.where((rpar == 0) & (cpar == 0), x, 0.0))

    ohr = jax.lax.broadcasted_iota(jnp.int32, (_B, _C), 0) == bi
    ohc = jax.lax.broadcasted_iota(jnp.int32, (_B, _C), 1) == ci
    o_ref[...] += jnp.where(ohr & ohc, s, 0.0)

    @pl.when((bi == _B - 1) & (ci == _C - 1))
    def _mlp():
        pooled = o_ref[...] * (1.0 / (_H * _W))
        h = jax.lax.dot_general(
            pooled, fc1_ref[...], (((1,), (1,)), ((), ())),
            preferred_element_type=jnp.float32)
        h = jnp.maximum(h, 0.0)
        o_ref[...] = jax.lax.dot_general(
            h, fc2_ref[...], (((1,), (1,)), ((), ())),
            preferred_element_type=jnp.float32)


def kernel(a, b, attn_w, attn_b, fc1_w, fc2_w):
    del a, attn_w, attn_b
    fc1 = fc1_w.reshape(_C // 4, _C)
    fc2 = fc2_w.reshape(_C, _C // 4)
    out = pl.pallas_call(
        _pool_mlp_body,
        grid=(_B, _C),
        in_specs=[
            pl.BlockSpec((1, 1, _H, _W), lambda i, j: (i, j, 0, 0)),
            pl.BlockSpec((_C // 4, _C), lambda i, j: (0, 0)),
            pl.BlockSpec((_C, _C // 4), lambda i, j: (0, 0)),
        ],
        out_specs=pl.BlockSpec((_B, _C), lambda i, j: (0, 0)),
        out_shape=jax.ShapeDtypeStruct((_B, _C), jnp.float32),
    )(b, fc1, fc2)
    return out.reshape(_B, _C, 1, 1)


# speedup vs baseline: 44.6958x; 2.8856x over previous
import functools

import jax
import jax.numpy as jnp
from jax import lax
from jax.experimental import pallas as pl
from jax.experimental.pallas import tpu as pltpu
from jax.experimental.pallas import tpu_sc as plsc

_B, _C, _H, _W = 4, 96, 384, 384
_NW = 32
_PLANES = _B * _C
_PPW = _PLANES // _NW
_CH_ROWS = 96
_CHUNKS = _PPW * 2
_NBUF = 3
_LANES = 16


def _sc_reduce_body(bt_hbm, out_hbm, idx0, idx1, idx2, buf0, buf1, buf2,
                    pacc_v, sem0, sem1, sem2):
    wid = lax.axis_index("s") * 2 + lax.axis_index("c")
    w12 = wid * _PPW
    liota = lax.iota(jnp.int32, _LANES)
    zeros = jnp.zeros((_LANES,), jnp.float32)

    slots = ((idx0, buf0, sem0), (idx1, buf1, sem1), (idx2, buf2, sem2))

    for j in range(_PPW):
        pacc_v[j] = zeros

    def fill_idx(idx_ref, k):
        plane = w12 + k // 2
        base = plane * _H + (k % 2) * (2 * _CH_ROWS)
        for j in range(_CH_ROWS // _LANES):
            idx_ref[pl.ds(j * _LANES, _LANES)] = (
                base + 2 * (j * _LANES) + 2 * liota)

    def start_gather(slot, k):
        idx_ref, buf_ref, sem = slot
        fill_idx(idx_ref, k)
        pltpu.make_async_copy(bt_hbm.at[idx_ref], buf_ref, sem).start()

    def consume(slot, k):
        idx_ref, buf_ref, sem = slot
        pltpu.make_async_copy(bt_hbm.at[idx_ref], buf_ref, sem).wait()

        def rbody(r, acc):
            for j in range(_W // _LANES):
                acc = acc + buf_ref[r, pl.ds(j * _LANES, _LANES)]
            return acc

        acc = lax.fori_loop(0, _CH_ROWS, rbody, zeros)
        pj = k // 2
        pacc_v[pj] = pacc_v[pj] + acc

    for b in range(_NBUF):
        start_gather(slots[b], jnp.int32(b))

    n_groups = _CHUNKS // _NBUF - 1

    def gbody(g, carry):
        for b in range(_NBUF):
            k = g * _NBUF + b
            consume(slots[b], k)
            start_gather(slots[b], k + _NBUF)
        return carry

    lax.fori_loop(0, n_groups, gbody, jnp.int32(0))

    for b in range(_NBUF):
        k = n_groups * _NBUF + b
        consume(slots[b], jnp.int32(k))

    pltpu.sync_copy(pacc_v, out_hbm.at[wid])


def _sc_pool_partials(b):
    bt = b.reshape(_PLANES * _H, _W)
    mesh = plsc.VectorSubcoreMesh(core_axis_name="c", subcore_axis_name="s")
    run = functools.partial(
        pl.kernel,
        out_type=jax.ShapeDtypeStruct((_NW, _PPW, _LANES), jnp.float32),
        mesh=mesh,
        scratch_types=[
            pltpu.VMEM((_CH_ROWS,), jnp.int32),
            pltpu.VMEM((_CH_ROWS,), jnp.int32),
            pltpu.VMEM((_CH_ROWS,), jnp.int32),
            pltpu.VMEM((_CH_ROWS, _W), jnp.float32),
            pltpu.VMEM((_CH_ROWS, _W), jnp.float32),
            pltpu.VMEM((_CH_ROWS, _W), jnp.float32),
            pltpu.VMEM((_PPW, _LANES), jnp.float32),
            pltpu.SemaphoreType.DMA,
            pltpu.SemaphoreType.DMA,
            pltpu.SemaphoreType.DMA,
        ],
    )(_sc_reduce_body)
    return run(bt)


def _mlp_body(part_ref, fc1_ref, fc2_ref, o_ref):
    part = part_ref[...]
    lane = lax.broadcasted_iota(jnp.int32, (_B, _C, _LANES), 2)
    pooled = jnp.sum(jnp.where(lane % 2 == 0, part, 0.0), axis=2)
    pooled = pooled * (1.0 / (_H * _W))
    h = lax.dot_general(pooled, fc1_ref[...], (((1,), (1,)), ((), ())),
                        preferred_element_type=jnp.float32)
    h = jnp.maximum(h, 0.0)
    o_ref[...] = lax.dot_general(h, fc2_ref[...], (((1,), (1,)), ((), ())),
                                 preferred_element_type=jnp.float32)


def kernel(a, b, attn_w, attn_b, fc1_w, fc2_w):
    del a, attn_w, attn_b
    partials = _sc_pool_partials(b).reshape(_B, _C, _LANES)
    fc1 = fc1_w.reshape(_C // 4, _C)
    fc2 = fc2_w.reshape(_C, _C // 4)
    out = pl.pallas_call(
        _mlp_body,
        out_shape=jax.ShapeDtypeStruct((_B, _C), jnp.float32),
    )(partials, fc1, fc2)
    return out.reshape(_B, _C, 1, 1)


# speedup vs baseline: 65.0589x; 1.4556x over previous
import functools

import jax
import jax.numpy as jnp
from jax import lax
from jax.experimental import pallas as pl
from jax.experimental.pallas import tpu as pltpu
from jax.experimental.pallas import tpu_sc as plsc

_B, _C, _H, _W = 4, 96, 384, 384
_NW = 32
_PLANES = _B * _C
_PPW = _PLANES // _NW
_CH_ROWS = 96
_CHUNKS = _PPW * 2
_NBUF = 3
_LANES = 16


def _sc_reduce_body(bt_hbm, out_hbm, idx0, idx1, idx2, buf0, buf1, buf2,
                    pacc_v, sem0, sem1, sem2):
    wid = lax.axis_index("s") * 2 + lax.axis_index("c")
    w12 = wid * _PPW
    liota = lax.iota(jnp.int32, _LANES)
    zeros = jnp.zeros((_LANES,), jnp.float32)

    slots = ((idx0, buf0, sem0), (idx1, buf1, sem1), (idx2, buf2, sem2))

    for j in range(_PPW):
        pacc_v[j] = zeros

    def fill_idx(idx_ref, k):
        plane = w12 + k // 2
        base = plane * _H + (k % 2) * (2 * _CH_ROWS)
        for j in range(_CH_ROWS // _LANES):
            idx_ref[pl.ds(j * _LANES, _LANES)] = (
                base + 2 * (j * _LANES) + 2 * liota)

    def start_gather(slot, k):
        idx_ref, buf_ref, sem = slot
        fill_idx(idx_ref, k)
        pltpu.make_async_copy(bt_hbm.at[idx_ref], buf_ref, sem).start()

    def consume(slot, k):
        idx_ref, buf_ref, sem = slot
        pltpu.make_async_copy(bt_hbm.at[idx_ref], buf_ref, sem).wait()

        def rbody(r, accs):
            accs = list(accs)
            for j in range(_W // _LANES):
                v = buf_ref[r, pl.ds(j * _LANES, _LANES)]
                accs[j % 4] = accs[j % 4] + v
            return tuple(accs)

        a0, a1, a2, a3 = lax.fori_loop(0, _CH_ROWS, rbody,
                                       (zeros, zeros, zeros, zeros))
        pj = k // 2
        pacc_v[pj] = pacc_v[pj] + ((a0 + a1) + (a2 + a3))

    for b in range(_NBUF):
        start_gather(slots[b], jnp.int32(b))

    n_groups = _CHUNKS // _NBUF - 1

    def gbody(g, carry):
        for b in range(_NBUF):
            k = g * _NBUF + b
            consume(slots[b], k)
            start_gather(slots[b], k + _NBUF)
        return carry

    lax.fori_loop(0, n_groups, gbody, jnp.int32(0))

    for b in range(_NBUF):
        k = n_groups * _NBUF + b
        consume(slots[b], jnp.int32(k))

    pltpu.sync_copy(pacc_v, out_hbm.at[wid])


def _sc_pool_partials(b):
    bt = b.reshape(_PLANES * _H, _W)
    mesh = plsc.VectorSubcoreMesh(core_axis_name="c", subcore_axis_name="s")
    run = functools.partial(
        pl.kernel,
        out_type=jax.ShapeDtypeStruct((_NW, _PPW, _LANES), jnp.float32),
        mesh=mesh,
        scratch_types=[
            pltpu.VMEM((_CH_ROWS,), jnp.int32),
            pltpu.VMEM((_CH_ROWS,), jnp.int32),
            pltpu.VMEM((_CH_ROWS,), jnp.int32),
            pltpu.VMEM((_CH_ROWS, _W), jnp.float32),
            pltpu.VMEM((_CH_ROWS, _W), jnp.float32),
            pltpu.VMEM((_CH_ROWS, _W), jnp.float32),
            pltpu.VMEM((_PPW, _LANES), jnp.float32),
            pltpu.SemaphoreType.DMA,
            pltpu.SemaphoreType.DMA,
            pltpu.SemaphoreType.DMA,
        ],
    )(_sc_reduce_body)
    return run(bt)


def _mlp_body(part_ref, fc1_ref, fc2_ref, o_ref):
    part = part_ref[...]
    lane = lax.broadcasted_iota(jnp.int32, (_B, _C, _LANES), 2)
    pooled = jnp.sum(jnp.where(lane % 2 == 0, part, 0.0), axis=2)
    pooled = pooled * (1.0 / (_H * _W))
    h = lax.dot_general(pooled, fc1_ref[...], (((1,), (1,)), ((), ())),
                        preferred_element_type=jnp.float32)
    h = jnp.maximum(h, 0.0)
    o_ref[...] = lax.dot_general(h, fc2_ref[...], (((1,), (1,)), ((), ())),
                                 preferred_element_type=jnp.float32)


def kernel(a, b, attn_w, attn_b, fc1_w, fc2_w):
    del a, attn_w, attn_b
    partials = _sc_pool_partials(b).reshape(_B, _C, _LANES)
    fc1 = fc1_w.reshape(_C // 4, _C)
    fc2 = fc2_w.reshape(_C, _C // 4)
    out = pl.pallas_call(
        _mlp_body,
        out_shape=jax.ShapeDtypeStruct((_B, _C), jnp.float32),
    )(partials, fc1, fc2)
    return out.reshape(_B, _C, 1, 1)
